# Initial kernel scaffold; baseline (speedup 1.0000x reference)
import jax
import jax.numpy as jnp
from jax.experimental import pallas as pl


def kernel(h, pos, edge_index, h_init, W_e1, b_e1, W_e2, b_e2, W_x1, b_x1, W_x2, b_x2, W_h1, b_h1, W_h2, b_h2):
    raise NotImplementedError("write your pallas kernel here")


# speedup vs baseline: 1.7149x; 1.7149x over previous
import functools

import jax
import jax.numpy as jnp
from jax import lax
from jax.experimental import pallas as pl
from jax.experimental.pallas import tpu as pltpu
from jax.experimental.pallas import tpu_sc as plsc

N = 10000
E = 160000
D = 256

NC = 2
NS = 16
L = 16

NP = 10240
EP = 163840
PADNODE = 10200

HALF = NP // 2
SENT = -1

MW = 384

GA = 128
NGA = (EP // 32) // GA
GC = 64
NGC = (EP // NS) // GC


def _silu(x):
    return x * jax.nn.sigmoid(x)


def _prep_body(h_ref, hi_ref, w1a_ref, w1b_ref, be1_ref, hin_ref, ad_ref, as_ref):
    h_in = h_ref[...] + hi_ref[...]
    hin_ref[...] = h_in
    ad_ref[...] = jnp.dot(h_in, w1a_ref[...], preferred_element_type=jnp.float32) + be1_ref[...]
    as_ref[...] = jnp.dot(h_in, w1b_ref[...], preferred_element_type=jnp.float32)


def _prep_call(hp, hip, w1a, w1b, be1, interpret=False):
    nb = NP // 256
    return pl.pallas_call(
        _prep_body,
        grid=(nb,),
        in_specs=[
            pl.BlockSpec((256, D), lambda i: (i, 0)),
            pl.BlockSpec((256, D), lambda i: (i, 0)),
            pl.BlockSpec((D, D), lambda i: (0, 0)),
            pl.BlockSpec((D, D), lambda i: (0, 0)),
            pl.BlockSpec((1, D), lambda i: (0, 0)),
        ],
        out_specs=[
            pl.BlockSpec((256, D), lambda i: (i, 0)),
            pl.BlockSpec((256, D), lambda i: (i, 0)),
            pl.BlockSpec((256, D), lambda i: (i, 0)),
        ],
        out_shape=[
            jax.ShapeDtypeStruct((NP, D), jnp.float32),
            jax.ShapeDtypeStruct((NP, D), jnp.float32),
            jax.ShapeDtypeStruct((NP, D), jnp.float32),
        ],
        interpret=interpret,
    )(hp, hip, w1a, w1b, be1)


def _edge_body(pre_ref, rel_ref, wrow_ref, we2_ref, be2_ref, wx1_ref, bx1_ref,
               wx2_ref, bx2_ref, mcat_ref):
    rel = rel_ref[...]
    d2 = jnp.sum(rel * rel, axis=1, keepdims=True)
    x = pre_ref[...] + d2 * wrow_ref[...]
    m1 = _silu(x)
    m2 = _silu(jnp.dot(m1, we2_ref[...], preferred_element_type=jnp.float32) + be2_ref[...])
    t = _silu(jnp.dot(m2, wx1_ref[...], preferred_element_type=jnp.float32) + bx1_ref[...])
    xw = jnp.sum(t * wx2_ref[...], axis=1, keepdims=True) + bx2_ref[...]
    nrows = rel.shape[0]
    mcat_ref[...] = jnp.concatenate(
        [m2, rel * xw, jnp.ones((nrows, 1), jnp.float32),
         jnp.zeros((nrows, MW - D - 5), jnp.float32)], axis=1)


def _edge_call(pre, rel4, wrow, we2, be2, wx1, bx1, wx2r, bx2, interpret=False):
    eb = 1024
    nb = EP // eb
    return pl.pallas_call(
        _edge_body,
        grid=(nb,),
        in_specs=[
            pl.BlockSpec((eb, D), lambda i: (i, 0)),
            pl.BlockSpec((eb, 4), lambda i: (i, 0)),
            pl.BlockSpec((1, D), lambda i: (0, 0)),
            pl.BlockSpec((D, D), lambda i: (0, 0)),
            pl.BlockSpec((1, D), lambda i: (0, 0)),
            pl.BlockSpec((D, D), lambda i: (0, 0)),
            pl.BlockSpec((1, D), lambda i: (0, 0)),
            pl.BlockSpec((1, D), lambda i: (0, 0)),
            pl.BlockSpec((1, 1), lambda i: (0, 0)),
        ],
        out_specs=[
            pl.BlockSpec((eb, MW), lambda i: (i, 0)),
        ],
        out_shape=[
            jax.ShapeDtypeStruct((EP, MW), jnp.float32),
        ],
        interpret=interpret,
    )(pre, rel4, wrow, we2, be2, wx1, bx1, wx2r, bx2)


def _node_body(hin_ref, acc_ref, pos_ref, wh1a_ref, wh1b_ref,
               bh1_ref, wh2_ref, bh2_ref, hout_ref, pout_ref):
    h_in = hin_ref[...]
    acc = acc_ref[...]
    agg = acc[:, :D]
    z = (jnp.dot(h_in, wh1a_ref[...], preferred_element_type=jnp.float32)
         + jnp.dot(agg, wh1b_ref[...], preferred_element_type=jnp.float32)
         + bh1_ref[...])
    upd = jnp.dot(_silu(z), wh2_ref[...], preferred_element_type=jnp.float32) + bh2_ref[...]
    hout_ref[...] = h_in + upd
    cnt = jnp.maximum(acc[:, D + 4:D + 5], 1.0)
    pout_ref[...] = pos_ref[...] + acc[:, D:D + 4] / cnt


def _node_call(hinp, acc, posp, wh1a, wh1b, bh1, wh2, bh2, interpret=False):
    nb = NP // 256
    return pl.pallas_call(
        _node_body,
        grid=(nb,),
        in_specs=[
            pl.BlockSpec((256, D), lambda i: (i, 0)),
            pl.BlockSpec((256, MW), lambda i: (i, 0)),
            pl.BlockSpec((256, 4), lambda i: (i, 0)),
            pl.BlockSpec((D, D), lambda i: (0, 0)),
            pl.BlockSpec((D, D), lambda i: (0, 0)),
            pl.BlockSpec((1, D), lambda i: (0, 0)),
            pl.BlockSpec((D, D), lambda i: (0, 0)),
            pl.BlockSpec((1, D), lambda i: (0, 0)),
        ],
        out_specs=[
            pl.BlockSpec((256, D), lambda i: (i, 0)),
            pl.BlockSpec((256, 4), lambda i: (i, 0)),
        ],
        out_shape=[
            jax.ShapeDtypeStruct((NP, D), jnp.float32),
            jax.ShapeDtypeStruct((NP, 4), jnp.float32),
        ],
        interpret=interpret,
    )(hinp, acc, posp, wh1a, wh1b, bh1, wh2, bh2)


def _sc_gather_body(ad_hbm, as_hbm, posf_hbm, dst_hbm, src_hbm,
                    pre_hbm, rel_hbm,
                    pos_v, rows_a, rows_b, dsti_v, srci_v, rel_v, sem_a, sem_b):
    wid = lax.axis_index("s") * NC + lax.axis_index("c")
    base = wid * (EP // 32)
    pltpu.sync_copy(posf_hbm, pos_v)
    lanes = lax.iota(jnp.int32, L)
    z16 = jnp.zeros((L,), jnp.float32)

    def group(g, _):
        eb = base + g * GA
        pltpu.sync_copy(dst_hbm.at[pl.ds(eb, GA)], dsti_v)
        pltpu.sync_copy(src_hbm.at[pl.ds(eb, GA)], srci_v)
        cpa = pltpu.async_copy(ad_hbm.at[dsti_v], rows_a, sem_a)
        cpb = pltpu.async_copy(as_hbm.at[srci_v], rows_b, sem_b)
        cpa.wait()
        cpb.wait()

        def sub(q, _):
            d16 = dsti_v[pl.ds(q * L, L)]
            s16 = srci_v[pl.ds(q * L, L)]
            d3 = d16 * 3
            s3 = s16 * 3
            rx = plsc.load_gather(pos_v, [d3]) - plsc.load_gather(pos_v, [s3])
            ry = plsc.load_gather(pos_v, [d3 + 1]) - plsc.load_gather(pos_v, [s3 + 1])
            rz = plsc.load_gather(pos_v, [d3 + 2]) - plsc.load_gather(pos_v, [s3 + 2])
            r4 = (q * L + lanes) * 4
            plsc.store_scatter(rel_v, [r4], rx)
            plsc.store_scatter(rel_v, [r4 + 1], ry)
            plsc.store_scatter(rel_v, [r4 + 2], rz)
            plsc.store_scatter(rel_v, [r4 + 3], z16)
            return 0

        lax.fori_loop(0, GA // L, sub, 0, unroll=True)

        def row(r, _):
            for k in range(D // L):
                sl = pl.ds(k * L, L)
                rows_a[r, sl] = rows_a[r, sl] + rows_b[r, sl]
            return 0

        lax.fori_loop(0, GA, row, 0)
        pltpu.sync_copy(rows_a, pre_hbm.at[pl.ds(eb, GA)])
        pltpu.sync_copy(rel_v, rel_hbm.at[pl.ds(eb * 4, GA * 4)])
        return 0

    lax.fori_loop(0, NGA, group, 0)


def _sc_gather_call(a_dst, a_src, posf, dstp, srcp):
    mesh = plsc.VectorSubcoreMesh(core_axis_name="c", subcore_axis_name="s")
    f = pl.kernel(
        _sc_gather_body,
        out_type=[
            jax.ShapeDtypeStruct((EP, D), jnp.float32),
            jax.ShapeDtypeStruct((EP * 4,), jnp.float32),
        ],
        mesh=mesh,
        scratch_types=[
            pltpu.VMEM((3 * NP,), jnp.float32),
            pltpu.VMEM((GA, D), jnp.float32),
            pltpu.VMEM((GA, D), jnp.float32),
            pltpu.VMEM((GA,), jnp.int32),
            pltpu.VMEM((GA,), jnp.int32),
            pltpu.VMEM((GA * 4,), jnp.float32),
            pltpu.SemaphoreType.DMA,
            pltpu.SemaphoreType.DMA,
        ],
        compiler_params=pltpu.CompilerParams(needs_layout_passes=False),
    )
    return f(a_dst, a_src, posf, dstp, srcp)


def _sc_segsum_body(mcat_hbm, dst_hbm, acc_hbm, rows, idx_v, dsti_v):
    c = lax.axis_index("c")
    sid = lax.axis_index("s")
    base_node = c * HALF

    z16 = jnp.zeros((L,), jnp.float32)

    def zrow(r, _):
        for k in range(MW // L):
            rows[r, pl.ds(k * L, L)] = z16
        return 0

    lax.fori_loop(0, GC, zrow, 0)

    stripe = HALF // NS
    r0 = base_node + sid * stripe
    off = 0
    for sz in ([GC] * (stripe // GC) + ([stripe % GC] if stripe % GC else [])):
        pltpu.sync_copy(rows.at[pl.ds(0, sz)], acc_hbm.at[pl.ds(r0 + off, sz)])
        off += sz
    plsc.subcore_barrier()

    lanes = lax.iota(jnp.int32, L)

    def group(g, _):
        eb = sid * (EP // NS) + g * GC
        pltpu.sync_copy(dst_hbm.at[pl.ds(eb, GC)], dsti_v)
        pltpu.sync_copy(mcat_hbm.at[pl.ds(eb, GC)], rows)

        def sub(q, _):
            d16 = dsti_v[pl.ds(q * L, L)]
            loc = d16 - base_node
            owned = (loc >= 0) & (loc < HALF) & (d16 < N)
            idx_v[pl.ds(q * L, L)] = jnp.where(owned, d16, N + q * L + lanes)
            return 0

        lax.fori_loop(0, GC // L, sub, 0, unroll=True)
        pltpu.sync_copy(rows, acc_hbm.at[idx_v], add=True)
        return 0

    lax.fori_loop(0, NGC, group, 0)


def _sc_segsum_call(mcat, dstp):
    mesh = plsc.VectorSubcoreMesh(core_axis_name="c", subcore_axis_name="s")
    f = pl.kernel(
        _sc_segsum_body,
        out_type=[
            jax.ShapeDtypeStruct((NP, MW), jnp.float32),
        ],
        mesh=mesh,
        scratch_types=[
            pltpu.VMEM((GC, MW), jnp.float32),
            pltpu.VMEM((GC,), jnp.int32),
            pltpu.VMEM((GC,), jnp.int32),
        ],
        compiler_params=pltpu.CompilerParams(needs_layout_passes=False),
    )
    return f(mcat, dstp)


def kernel(h, pos, edge_index, h_init, W_e1, b_e1, W_e2, b_e2,
           W_x1, b_x1, W_x2, b_x2, W_h1, b_h1, W_h2, b_h2):
    f32 = jnp.float32
    hp = jnp.pad(h, ((0, NP - N), (0, 0)))
    hip = jnp.pad(h_init, ((0, NP - N), (0, 0)))
    posf = jnp.pad(pos, ((0, NP - N), (0, 0))).reshape(3 * NP)
    src = edge_index[0]
    dst = edge_index[1]
    padi = jnp.full((EP - E,), PADNODE, jnp.int32)
    srcp = jnp.concatenate([src, padi])
    dstp = jnp.concatenate([dst, padi])

    w1a = W_e1[:D]
    w1b = W_e1[D:2 * D]
    wrow = W_e1[2 * D:2 * D + 1]
    be1 = b_e1.reshape(1, D)

    hinp, a_dst, a_src = _prep_call(hp, hip, w1a, w1b, be1)
    pre, relf = _sc_gather_call(a_dst, a_src, posf, dstp, srcp)
    (mcat,) = _edge_call(pre, relf.reshape(EP, 4), wrow, W_e2, b_e2.reshape(1, D),
                         W_x1, b_x1.reshape(1, D), W_x2.reshape(1, D),
                         b_x2.reshape(1, 1))
    (acc,) = _sc_segsum_call(mcat, dstp)
    posp4 = jnp.pad(pos, ((0, NP - N), (0, 1)))

    hout, pout = _node_call(hinp, acc, posp4,
                            W_h1[:D], W_h1[D:], b_h1.reshape(1, D),
                            W_h2, b_h2.reshape(1, D))
    return (hout[:N], pout[:N, :3].astype(f32))
